# baseline (speedup 1.0000x reference)
import functools

import jax
import jax.numpy as jnp
from jax import lax
from jax.experimental import pallas as pl
from jax.experimental.pallas import tpu as pltpu
from jax.experimental.pallas import tpu_sc as plsc

N_CAT = 1000
BATCH = 16384
NUM_CORES = 2
NUM_SUBCORES = 16
LANES = 16
NUM_WORKERS = NUM_CORES * NUM_SUBCORES
ROWS_PER_W = BATCH // NUM_WORKERS
CHUNK_ROWS = 32
CHUNK_WORDS = CHUNK_ROWS * N_CAT
NUM_CHUNKS = ROWS_PER_W // CHUNK_ROWS

_mesh = plsc.VectorSubcoreMesh(core_axis_name="c", subcore_axis_name="s")


@functools.partial(
    pl.kernel,
    out_type=jax.ShapeDtypeStruct((BATCH * N_CAT,), jnp.float32),
    mesh=_mesh,
    compiler_params=pltpu.CompilerParams(needs_layout_passes=False),
    scratch_types=[
        pltpu.VMEM((ROWS_PER_W,), jnp.int32),
        pltpu.VMEM((CHUNK_WORDS,), jnp.float32),
        pltpu.VMEM((CHUNK_WORDS,), jnp.float32),
        pltpu.SemaphoreType.DMA,
        pltpu.SemaphoreType.DMA,
    ],
)
def _onehot_sc(x_hbm, out_hbm, idx_v, buf_a, buf_b, sem_a, sem_b):
    wid = lax.axis_index("s") * NUM_CORES + lax.axis_index("c")
    base_row = wid * ROWS_PER_W

    pltpu.sync_copy(x_hbm.at[pl.ds(base_row * 1, ROWS_PER_W)], idx_v)

    zeros = jnp.zeros((LANES,), jnp.float32)
    ones = jnp.ones((LANES,), jnp.float32)
    lane_off = lax.iota(jnp.int32, LANES) * N_CAT

    def _zero(i, _):
        buf_a[pl.ds(i * LANES, LANES)] = zeros
        buf_b[pl.ds(i * LANES, LANES)] = zeros
        return 0

    lax.fori_loop(0, CHUNK_WORDS // LANES, _zero, 0, unroll=8)

    bufs = (buf_a, buf_b)
    sems = (sem_a, sem_b)

    def chunk_flat_indices(g):
        flats = []
        for t in range(CHUNK_ROWS // LANES):
            xv = idx_v[pl.ds(g * CHUNK_ROWS + t * LANES, LANES)]
            flats.append(lane_off + (t * LANES * N_CAT) + xv)
        return flats

    inflight = [None, None]
    for g in range(NUM_CHUNKS):
        b = g % 2
        buf = bufs[b]
        if inflight[b] is not None:
            copy, old_flats = inflight[b]
            copy.wait()
            for fv in old_flats:
                plsc.store_scatter(buf, [fv], zeros)
        flats = chunk_flat_indices(g)
        for fv in flats:
            plsc.store_scatter(buf, [fv], ones)
        dst = out_hbm.at[pl.ds((base_row + g * CHUNK_ROWS) * N_CAT, CHUNK_WORDS)]
        copy = pltpu.async_copy(buf, dst, sems[b])
        inflight[b] = (copy, flats)

    for b in range(2):
        if inflight[b] is not None:
            inflight[b][0].wait()


def kernel(x):
    out_flat = _onehot_sc(x.astype(jnp.int32))
    return out_flat.reshape(BATCH, N_CAT)
